# baseline (speedup 1.0000x reference)
import functools

import jax
import jax.numpy as jnp
from jax import lax
from jax.experimental import pallas as pl
from jax.experimental.pallas import tpu as pltpu
from jax.experimental.pallas import tpu_sc as plsc

NW = 32
L = 16


def _make_gather(nb: int, nt: int, dim: int):
    nbt = nb // 128
    ndt = dim // 8
    dts_per_w = (ndt + NW - 1) // NW
    mesh = plsc.VectorSubcoreMesh(core_axis_name="c", subcore_axis_name="s")

    @functools.partial(
        pl.kernel,
        mesh=mesh,
        out_type=jax.ShapeDtypeStruct((nt, ndt, nbt, 8, 128), jnp.float32),
        scratch_types=[
            pltpu.VMEM((nb * nt,), jnp.int32),
            pltpu.VMEM((8 * dts_per_w * dim,), jnp.float32),
            pltpu.VMEM((2, nbt, 8, 128), jnp.float32),
            pltpu.SemaphoreType.DMA,
            pltpu.SemaphoreType.DMA,
        ],
        compiler_params=pltpu.CompilerParams(use_tc_tiling_on_sc=False,
                                             needs_layout_passes=False),
    )
    def gather_kernel(idxt_hbm, tablet_flat_hbm, out_hbm, idx_l, stripe,
                      obuf, wsem0, wsem1):
        wid = lax.axis_index("s") * 2 + lax.axis_index("c")

        pltpu.sync_copy(idxt_hbm, idx_l)
        for i in range(dts_per_w):
            dt = wid + NW * i

            @pl.when(dt < ndt)
            def _():
                pltpu.sync_copy(tablet_flat_hbm.at[pl.ds(dt * 8 * dim, 8 * dim)],
                                stripe.at[pl.ds(8 * i * dim, 8 * dim)])

        def wait_put(p, wsem):
            pltpu.make_async_copy(obuf.at[p], out_hbm.at[0, 0], wsem).wait()

        for i in range(dts_per_w):
            dt = wid + NW * i

            @pl.when(dt < ndt)
            def _():
                def t_body(t, carry):
                    p = lax.rem(t, 2)
                    not_first = i * nt + t >= 2

                    @pl.when(jnp.logical_and(p == 0, not_first))
                    def _():
                        wait_put(0, wsem0)

                    @pl.when(jnp.logical_and(p == 1, not_first))
                    def _():
                        wait_put(1, wsem1)

                    for bt in range(nbt):
                        for g in range(8):
                            idx16 = idx_l[pl.ds(t * nb + bt * 128 + L * g, L)]
                            vals = [
                                plsc.load_gather(
                                    stripe,
                                    [idx16 + jnp.int32((8 * i + dl) * dim)])
                                for dl in range(8)
                            ]
                            for dl in range(8):
                                obuf[p, bt, dl, pl.ds(L * g, L)] = vals[dl]

                    @pl.when(p == 0)
                    def _():
                        pltpu.async_copy(obuf.at[0], out_hbm.at[t, dt], wsem0)

                    @pl.when(p == 1)
                    def _():
                        pltpu.async_copy(obuf.at[1], out_hbm.at[t, dt], wsem1)

                    return carry

                lax.fori_loop(0, nt, t_body, 0)

        wait_put(0, wsem0)
        wait_put(1, wsem1)

    return gather_kernel


def kernel(input, table):
    nb, nt = input.shape
    vocab, dim = table.shape
    idxt = input.T.astype(jnp.int32).reshape(-1)
    tablet = table.T.reshape(-1)
    out5 = _make_gather(nb, nt, dim)(idxt, tablet)
    return out5.transpose(2, 4, 0, 1, 3).reshape(nb, nt, dim)


# speedup vs baseline: 1.1049x; 1.1049x over previous
import functools

import jax
import jax.numpy as jnp
from jax import lax
from jax.experimental import pallas as pl
from jax.experimental.pallas import tpu as pltpu
from jax.experimental.pallas import tpu_sc as plsc

NW = 32
L = 16


def _make_gather(nb: int, nt: int, dim: int):
    nbt = nb // 128
    ndt = dim // 8
    dts_per_w = (ndt + NW - 1) // NW
    mesh = plsc.VectorSubcoreMesh(core_axis_name="c", subcore_axis_name="s")

    @functools.partial(
        pl.kernel,
        mesh=mesh,
        out_type=jax.ShapeDtypeStruct((nt, ndt, nbt, 8, 128), jnp.float32),
        scratch_types=[
            pltpu.VMEM((nb * nt,), jnp.int32),
            pltpu.VMEM((8 * dts_per_w * dim,), jnp.float32),
            pltpu.VMEM((8, 128), jnp.float32),
            pltpu.VMEM((8, 128), jnp.float32),
            pltpu.SemaphoreType.DMA,
            pltpu.SemaphoreType.DMA,
        ],
        compiler_params=pltpu.CompilerParams(use_tc_tiling_on_sc=False,
                                             needs_layout_passes=False),
    )
    def gather_kernel(idxt_hbm, tablet_flat_hbm, out_hbm, idx_l, stripe,
                      ob0, ob1, wsem0, wsem1):
        wid = lax.axis_index("s") * 2 + lax.axis_index("c")

        pltpu.sync_copy(idxt_hbm, idx_l)
        for i in range(dts_per_w):
            dt = wid + NW * i

            @pl.when(dt < ndt)
            def _():
                pltpu.sync_copy(tablet_flat_hbm.at[pl.ds(dt * 8 * dim, 8 * dim)],
                                stripe.at[pl.ds(8 * i * dim, 8 * dim)])

        def wait_put(ob, wsem):
            pltpu.make_async_copy(ob, out_hbm.at[0, 0, 0], wsem).wait()

        def gather_batch(i, t, bt, g):
            idx16 = idx_l[pl.ds(t * nb + bt * 128 + L * g, L)]
            return [
                plsc.load_gather(stripe,
                                 [idx16 + jnp.int32((8 * i + dl) * dim)])
                for dl in range(8)
            ]

        def emit_tile(i, t, bt, first, ob, wsem):
            dt = wid + NW * i

            @pl.when(jnp.logical_not(first))
            def _():
                wait_put(ob, wsem)

            vals = gather_batch(i, t, bt, 0)
            for g in range(8):
                nxt = gather_batch(i, t, bt, g + 1) if g < 7 else None
                for dl in range(8):
                    ob[dl, pl.ds(L * g, L)] = vals[dl]
                vals = nxt
            pltpu.async_copy(ob, out_hbm.at[t, dt, bt], wsem)

        for i in range(dts_per_w):
            dt = wid + NW * i

            @pl.when(dt < ndt)
            def _():
                def t_body(t, carry):
                    def bt_body(btp, carry2):
                        n = ((i * nt + t) * (nbt // 2) + btp) > 0
                        emit_tile(i, t, 2 * btp, jnp.logical_not(n), ob0, wsem0)
                        emit_tile(i, t, 2 * btp + 1, jnp.logical_not(n), ob1, wsem1)
                        return carry2

                    return lax.fori_loop(0, nbt // 2, bt_body, carry)

                lax.fori_loop(0, nt, t_body, 0)

        wait_put(ob0, wsem0)
        wait_put(ob1, wsem1)

    return gather_kernel


def kernel(input, table):
    nb, nt = input.shape
    vocab, dim = table.shape
    idxt = input.T.astype(jnp.int32).reshape(-1)
    tablet = table.T.reshape(-1)
    out5 = _make_gather(nb, nt, dim)(idxt, tablet)
    return out5.transpose(2, 4, 0, 1, 3).reshape(nb, nt, dim)


# speedup vs baseline: 1.4443x; 1.3072x over previous
import functools

import jax
import jax.numpy as jnp
from jax import lax
from jax.experimental import pallas as pl
from jax.experimental.pallas import tpu as pltpu
from jax.experimental.pallas import tpu_sc as plsc

NW = 32
L = 16


def _make_gather(nb: int, nt: int, dim: int):
    nbt = nb // 128
    ndt = dim // 8
    dts_per_w = (ndt + NW - 1) // NW
    mesh = plsc.VectorSubcoreMesh(core_axis_name="c", subcore_axis_name="s")

    @functools.partial(
        pl.kernel,
        mesh=mesh,
        out_type=jax.ShapeDtypeStruct((nt, ndt, nbt, 8, 128), jnp.float32),
        scratch_types=[
            pltpu.VMEM((nb * nt,), jnp.int32),
            pltpu.VMEM((8 * dts_per_w * dim,), jnp.float32),
            pltpu.VMEM((8, 128), jnp.float32),
            pltpu.VMEM((8, 128), jnp.float32),
            pltpu.SemaphoreType.DMA,
            pltpu.SemaphoreType.DMA,
        ],
        compiler_params=pltpu.CompilerParams(use_tc_tiling_on_sc=False,
                                             needs_layout_passes=False),
    )
    def gather_kernel(idxt_hbm, tablet_flat_hbm, out_hbm, idx_l, stripe,
                      ob0, ob1, wsem0, wsem1):
        wid = lax.axis_index("s") * 2 + lax.axis_index("c")

        pltpu.sync_copy(idxt_hbm, idx_l)
        for i in range(dts_per_w):
            dt = wid + NW * i

            @pl.when(dt < ndt)
            def _():
                pltpu.sync_copy(tablet_flat_hbm.at[pl.ds(dt * 8 * dim, 8 * dim)],
                                stripe.at[pl.ds(8 * i * dim, 8 * dim)])

        def wait_put(ob, wsem):
            pltpu.make_async_copy(ob, out_hbm.at[0, 0, 0], wsem).wait()

        def gather_batch(i, t, bt, g):
            idx16 = idx_l[pl.ds(t * nb + bt * 128 + L * g, L)]
            return [
                plsc.load_gather(stripe,
                                 [idx16 + jnp.int32((8 * i + dl) * dim)])
                for dl in range(8)
            ]

        def emit_tile(i, t, bt, first, ob, wsem):
            dt = wid + NW * i

            @pl.when(jnp.logical_not(first))
            def _():
                wait_put(ob, wsem)

            @plsc.parallel_loop(0, 8, unroll=8)
            def _(g):
                vals = gather_batch(i, t, bt, g)
                for dl in range(8):
                    ob[dl, pl.ds(L * g, L)] = vals[dl]

            pltpu.async_copy(ob, out_hbm.at[t, dt, bt], wsem)

        for i in range(dts_per_w):
            dt = wid + NW * i

            @pl.when(dt < ndt)
            def _():
                def t_body(t, carry):
                    def bt_body(btp, carry2):
                        n = ((i * nt + t) * (nbt // 2) + btp) > 0
                        emit_tile(i, t, 2 * btp, jnp.logical_not(n), ob0, wsem0)
                        emit_tile(i, t, 2 * btp + 1, jnp.logical_not(n), ob1, wsem1)
                        return carry2

                    return lax.fori_loop(0, nbt // 2, bt_body, carry)

                lax.fori_loop(0, nt, t_body, 0)

        wait_put(ob0, wsem0)
        wait_put(ob1, wsem1)

    return gather_kernel


def kernel(input, table):
    nb, nt = input.shape
    vocab, dim = table.shape
    idxt = input.T.astype(jnp.int32).reshape(-1)
    tablet = table.T.reshape(-1)
    out5 = _make_gather(nb, nt, dim)(idxt, tablet)
    return out5.transpose(2, 4, 0, 1, 3).reshape(nb, nt, dim)
